# baseline (speedup 1.0000x reference)
import functools

import jax
import jax.numpy as jnp
from jax import lax
from jax.experimental import pallas as pl
from jax.experimental.pallas import tpu as pltpu
from jax.experimental.pallas import tpu_sc as plsc


def _emb_kernel(B, D, b_per_w, C, n_chunks):
    mesh = plsc.VectorSubcoreMesh(core_axis_name="c", subcore_axis_name="s")

    @functools.partial(
        pl.kernel,
        mesh=mesh,
        out_type=jax.ShapeDtypeStruct((B, D), jnp.float32),
        scratch_types=[
            pltpu.VMEM((b_per_w,), jnp.int32),
            pltpu.VMEM((C, D), jnp.float32),
            pltpu.VMEM((C, D), jnp.float32),
            pltpu.SemaphoreType.DMA,
            pltpu.SemaphoreType.DMA,
            pltpu.SemaphoreType.DMA,
            pltpu.SemaphoreType.DMA,
        ],
        compiler_params=pltpu.CompilerParams(use_tc_tiling_on_sc=False),
    )
    def emb(table_hbm, idx_hbm, out_hbm, idx_v, rows0, rows1, g0, g1, s0, s1):
        nc = 2
        wid = lax.axis_index("s") * nc + lax.axis_index("c")
        base = wid * b_per_w
        pltpu.sync_copy(idx_hbm.at[pl.ds(base, b_per_w)], idx_v)

        bufs = (rows0, rows1)
        gsems = (g0, g1)
        ssems = (s0, s1)

        def gather(ci):
            b = ci % 2
            return pltpu.async_copy(
                table_hbm.at[idx_v.at[pl.ds(ci * C, C)]], bufs[b], gsems[b]
            )

        def scatter(ci):
            b = ci % 2
            return pltpu.async_copy(
                bufs[b], out_hbm.at[pl.ds(base + ci * C, C)], ssems[b]
            )

        gathers = [None] * n_chunks
        scatters = [None] * n_chunks
        gathers[0] = gather(0)
        for ci in range(n_chunks):
            if ci + 1 < n_chunks:
                if ci - 1 >= 0:
                    scatters[ci - 1].wait()
                gathers[ci + 1] = gather(ci + 1)
            gathers[ci].wait()
            scatters[ci] = scatter(ci)
        scatters[n_chunks - 2].wait()
        scatters[n_chunks - 1].wait()

    return emb


def kernel(params, ids):
    V, D = params.shape
    ids_shape = ids.shape
    B = 1
    for s in ids_shape:
        B *= s
    NW = 32
    b_per_w = B // NW
    C = 800
    n_chunks = b_per_w // C

    ids_flat = ids.reshape((B,)).astype(jnp.int32)
    out = _emb_kernel(B, D, b_per_w, C, n_chunks)(params, ids_flat)
    return out.reshape(tuple(ids_shape) + (D,))


# speedup vs baseline: 1.2781x; 1.2781x over previous
import functools

import jax
import jax.numpy as jnp
from jax import lax
from jax.experimental import pallas as pl
from jax.experimental.pallas import tpu as pltpu
from jax.experimental.pallas import tpu_sc as plsc


def _transpose_kernel(x_ref, o_ref):
    x = x_ref[...]
    t = x.T
    bn = t.shape[0]
    t3 = t.reshape(bn // 2, 2, 64)
    o_ref[:, 0:64] = t3[:, 0, :]
    o_ref[:, 64:128] = t3[:, 1, :]


def _transpose_table(params_t, V, D, BN=4096):
    grid = pl.cdiv(V, BN)
    return pl.pallas_call(
        _transpose_kernel,
        grid=(grid,),
        in_specs=[pl.BlockSpec((D, BN), lambda b: (0, b))],
        out_specs=pl.BlockSpec((BN // 2, 2 * D), lambda b: (b, 0)),
        out_shape=jax.ShapeDtypeStruct((V // 2, 2 * D), jnp.float32),
        compiler_params=pltpu.CompilerParams(
            vmem_limit_bytes=100 * 1024 * 1024
        ),
    )(params_t)


def _emb_kernel(B, D, b_per_w, C, n_chunks):
    mesh = plsc.VectorSubcoreMesh(core_axis_name="c", subcore_axis_name="s")

    @functools.partial(
        pl.kernel,
        mesh=mesh,
        out_type=jax.ShapeDtypeStruct((B, D), jnp.float32),
        scratch_types=[
            pltpu.VMEM((b_per_w,), jnp.int32),
            pltpu.VMEM((C, D), jnp.float32),
            pltpu.VMEM((C, D), jnp.float32),
            pltpu.SemaphoreType.DMA,
            pltpu.SemaphoreType.DMA,
            pltpu.SemaphoreType.DMA,
            pltpu.SemaphoreType.DMA,
        ],
        compiler_params=pltpu.CompilerParams(use_tc_tiling_on_sc=False),
    )
    def emb(table_hbm, idx_hbm, out_hbm, idx_v, rows0, rows1, g0, g1, s0, s1):
        nc = 2
        wid = lax.axis_index("s") * nc + lax.axis_index("c")
        base = wid * b_per_w
        pltpu.sync_copy(idx_hbm.at[pl.ds(base, b_per_w)], idx_v)

        bufs = (rows0, rows1)
        gsems = (g0, g1)
        ssems = (s0, s1)

        def gather(ci):
            b = ci % 2
            return pltpu.async_copy(
                table_hbm.at[idx_v.at[pl.ds(ci * C, C)]], bufs[b], gsems[b]
            )

        def scatter(ci):
            b = ci % 2
            return pltpu.async_copy(
                bufs[b], out_hbm.at[pl.ds(base + ci * C, C)], ssems[b]
            )

        gathers = [None] * n_chunks
        scatters = [None] * n_chunks
        gathers[0] = gather(0)
        for ci in range(n_chunks):
            if ci + 1 < n_chunks:
                if ci - 1 >= 0:
                    scatters[ci - 1].wait()
                gathers[ci + 1] = gather(ci + 1)
            gathers[ci].wait()
            scatters[ci] = scatter(ci)
        scatters[n_chunks - 2].wait()
        scatters[n_chunks - 1].wait()

    return emb


def kernel(params, ids):
    V, D = params.shape
    ids_shape = ids.shape
    B = 1
    for s in ids_shape:
        B *= s
    NW = 32
    b_per_w = B // NW
    C = 800
    n_chunks = b_per_w // C

    packed = _transpose_table(params.T, V, D, BN=4096)
    table = packed.reshape(V, D)
    ids_flat = ids.reshape((B,)).astype(jnp.int32)
    out = _emb_kernel(B, D, b_per_w, C, n_chunks)(table, ids_flat)
    return out.reshape(tuple(ids_shape) + (D,))


# speedup vs baseline: 1.3655x; 1.0683x over previous
import functools

import jax
import jax.numpy as jnp
from jax import lax
from jax.experimental import pallas as pl
from jax.experimental.pallas import tpu as pltpu
from jax.experimental.pallas import tpu_sc as plsc

_BN = 2048
_H = _BN // 2


def _transpose_kernel(x_ref, o_ref):
    x = x_ref[...]
    xx = jnp.concatenate([x[:, :_H], x[:, _H:]], axis=0)
    o_ref[...] = xx.T


def _transpose_table(params_t, V, D):
    grid = pl.cdiv(V, _BN)
    return pl.pallas_call(
        _transpose_kernel,
        grid=(grid,),
        in_specs=[pl.BlockSpec((D, _BN), lambda b: (0, b))],
        out_specs=pl.BlockSpec((_H, 2 * D), lambda b: (b, 0)),
        out_shape=jax.ShapeDtypeStruct((grid * _H, 2 * D), jnp.float32),
        compiler_params=pltpu.CompilerParams(
            vmem_limit_bytes=100 * 1024 * 1024
        ),
    )(params_t)


def _emb_kernel(B, D, b_per_w, C, n_chunks):
    mesh = plsc.VectorSubcoreMesh(core_axis_name="c", subcore_axis_name="s")

    @functools.partial(
        pl.kernel,
        mesh=mesh,
        out_type=jax.ShapeDtypeStruct((B, D), jnp.float32),
        scratch_types=[
            pltpu.VMEM((b_per_w,), jnp.int32),
            pltpu.VMEM((C, D), jnp.float32),
            pltpu.VMEM((C, D), jnp.float32),
            pltpu.SemaphoreType.DMA,
            pltpu.SemaphoreType.DMA,
            pltpu.SemaphoreType.DMA,
            pltpu.SemaphoreType.DMA,
        ],
        compiler_params=pltpu.CompilerParams(use_tc_tiling_on_sc=False),
    )
    def emb(table_hbm, idx_hbm, out_hbm, idx_v, rows0, rows1, g0, g1, s0, s1):
        nc = 2
        wid = lax.axis_index("s") * nc + lax.axis_index("c")
        base = wid * b_per_w
        pltpu.sync_copy(idx_hbm.at[pl.ds(base, b_per_w)], idx_v)

        def remap(k, _):
            v = idx_v[pl.ds(k * 16, 16)]
            j = (
                (v & jnp.int32(~(_BN - 1)))
                | ((v & jnp.int32(_H - 1)) << 1)
                | ((v >> 10) & jnp.int32(1))
            )
            idx_v[pl.ds(k * 16, 16)] = j
            return 0

        lax.fori_loop(0, b_per_w // 16, remap, 0)

        bufs = (rows0, rows1)
        gsems = (g0, g1)
        ssems = (s0, s1)

        def gather(ci):
            b = ci % 2
            return pltpu.async_copy(
                table_hbm.at[idx_v.at[pl.ds(ci * C, C)]], bufs[b], gsems[b]
            )

        def scatter(ci):
            b = ci % 2
            return pltpu.async_copy(
                bufs[b], out_hbm.at[pl.ds(base + ci * C, C)], ssems[b]
            )

        gathers = [None] * n_chunks
        scatters = [None] * n_chunks
        gathers[0] = gather(0)
        for ci in range(n_chunks):
            if ci + 1 < n_chunks:
                if ci - 1 >= 0:
                    scatters[ci - 1].wait()
                gathers[ci + 1] = gather(ci + 1)
            gathers[ci].wait()
            scatters[ci] = scatter(ci)
        scatters[n_chunks - 2].wait()
        scatters[n_chunks - 1].wait()

    return emb


def kernel(params, ids):
    V, D = params.shape
    ids_shape = ids.shape
    B = 1
    for s in ids_shape:
        B *= s
    NW = 32
    b_per_w = B // NW
    C = 800
    n_chunks = b_per_w // C

    packed = _transpose_table(params.T, V, D)
    table = packed.reshape(packed.shape[0] * 2, D)
    ids_flat = ids.reshape((B,)).astype(jnp.int32)
    out = _emb_kernel(B, D, b_per_w, C, n_chunks)(table, ids_flat)
    return out.reshape(tuple(ids_shape) + (D,))


# speedup vs baseline: 1.7091x; 1.2516x over previous
import functools

import jax
import jax.numpy as jnp
from jax import lax
from jax.experimental import pallas as pl
from jax.experimental.pallas import tpu as pltpu
from jax.experimental.pallas import tpu_sc as plsc

_BN = 4096
_H = _BN // 2


def _transpose_kernel(x_ref, o_ref):
    x = x_ref[...]
    xx = jnp.concatenate([x[:, :_H], x[:, _H:]], axis=0)
    o_ref[...] = xx.T


def _transpose_table(params_t, V, D):
    grid = pl.cdiv(V, _BN)
    return pl.pallas_call(
        _transpose_kernel,
        grid=(grid,),
        in_specs=[pl.BlockSpec((D, _BN), lambda b: (0, b))],
        out_specs=pl.BlockSpec((_H, 2 * D), lambda b: (b, 0)),
        out_shape=jax.ShapeDtypeStruct((grid * _H, 2 * D), jnp.float32),
        compiler_params=pltpu.CompilerParams(
            vmem_limit_bytes=100 * 1024 * 1024
        ),
    )(params_t)


def _emb_kernel(B, D, b_per_w, C, n_chunks):
    mesh = plsc.VectorSubcoreMesh(core_axis_name="c", subcore_axis_name="s")

    @functools.partial(
        pl.kernel,
        mesh=mesh,
        out_type=jax.ShapeDtypeStruct((B, D), jnp.float32),
        scratch_types=[
            pltpu.VMEM((b_per_w,), jnp.int32),
            pltpu.VMEM((C, D), jnp.float32),
            pltpu.VMEM((C, D), jnp.float32),
            pltpu.SemaphoreType.DMA,
            pltpu.SemaphoreType.DMA,
            pltpu.SemaphoreType.DMA,
            pltpu.SemaphoreType.DMA,
        ],
        compiler_params=pltpu.CompilerParams(use_tc_tiling_on_sc=False),
    )
    def emb(table_hbm, idx_hbm, out_hbm, idx_v, rows0, rows1, g0, g1, s0, s1):
        nc = 2
        wid = lax.axis_index("s") * nc + lax.axis_index("c")
        base = wid * b_per_w
        pltpu.sync_copy(idx_hbm.at[pl.ds(base, b_per_w)], idx_v)

        def remap(k, _):
            v = idx_v[pl.ds(k * 16, 16)]
            j = (
                (v & jnp.int32(~(_BN - 1)))
                | ((v & jnp.int32(_H - 1)) << 1)
                | ((v >> (_H.bit_length() - 1)) & jnp.int32(1))
            )
            idx_v[pl.ds(k * 16, 16)] = j
            return 0

        lax.fori_loop(0, b_per_w // 16, remap, 0)

        bufs = (rows0, rows1)
        gsems = (g0, g1)
        ssems = (s0, s1)

        def gather(ci):
            b = ci % 2
            return pltpu.async_copy(
                table_hbm.at[idx_v.at[pl.ds(ci * C, C)]], bufs[b], gsems[b]
            )

        def scatter(ci):
            b = ci % 2
            return pltpu.async_copy(
                bufs[b], out_hbm.at[pl.ds(base + ci * C, C)], ssems[b]
            )

        gathers = [None] * n_chunks
        scatters = [None] * n_chunks
        gathers[0] = gather(0)
        for ci in range(n_chunks):
            if ci + 1 < n_chunks:
                if ci - 1 >= 0:
                    scatters[ci - 1].wait()
                gathers[ci + 1] = gather(ci + 1)
            gathers[ci].wait()
            scatters[ci] = scatter(ci)
        scatters[n_chunks - 2].wait()
        scatters[n_chunks - 1].wait()

    return emb


def kernel(params, ids):
    V, D = params.shape
    ids_shape = ids.shape
    B = 1
    for s in ids_shape:
        B *= s
    NW = 32
    b_per_w = B // NW
    C = 800
    n_chunks = b_per_w // C

    packed = _transpose_table(params.T, V, D)
    table = packed.reshape(packed.shape[0] * 2, D)
    ids_flat = ids.reshape((B,)).astype(jnp.int32)
    out = _emb_kernel(B, D, b_per_w, C, n_chunks)(table, ids_flat)
    return out.reshape(tuple(ids_shape) + (D,))


# speedup vs baseline: 2.0283x; 1.1868x over previous
import functools

import jax
import jax.numpy as jnp
from jax import lax
from jax.experimental import pallas as pl
from jax.experimental.pallas import tpu as pltpu
from jax.experimental.pallas import tpu_sc as plsc

_BN = 8192
_H = _BN // 2


def _transpose_kernel(x_ref, o_ref):
    x = x_ref[...]
    xx = jnp.concatenate([x[:, :_H], x[:, _H:]], axis=0)
    o_ref[...] = xx.T


def _transpose_table(params_t, V, D):
    grid = pl.cdiv(V, _BN)
    return pl.pallas_call(
        _transpose_kernel,
        grid=(grid,),
        in_specs=[pl.BlockSpec((D, _BN), lambda b: (0, b))],
        out_specs=pl.BlockSpec((_H, 2 * D), lambda b: (b, 0)),
        out_shape=jax.ShapeDtypeStruct((grid * _H, 2 * D), jnp.float32),
        compiler_params=pltpu.CompilerParams(
            vmem_limit_bytes=100 * 1024 * 1024
        ),
    )(params_t)


def _emb_kernel(B, D, b_per_w, C, n_chunks):
    mesh = plsc.VectorSubcoreMesh(core_axis_name="c", subcore_axis_name="s")

    @functools.partial(
        pl.kernel,
        mesh=mesh,
        out_type=jax.ShapeDtypeStruct((B, D), jnp.float32),
        scratch_types=[
            pltpu.VMEM((b_per_w,), jnp.int32),
            pltpu.VMEM((C, D), jnp.float32),
            pltpu.VMEM((C, D), jnp.float32),
            pltpu.SemaphoreType.DMA,
            pltpu.SemaphoreType.DMA,
            pltpu.SemaphoreType.DMA,
            pltpu.SemaphoreType.DMA,
        ],
        compiler_params=pltpu.CompilerParams(use_tc_tiling_on_sc=False),
    )
    def emb(table_hbm, idx_hbm, out_hbm, idx_v, rows0, rows1, g0, g1, s0, s1):
        nc = 2
        wid = lax.axis_index("s") * nc + lax.axis_index("c")
        base = wid * b_per_w
        pltpu.sync_copy(idx_hbm.at[pl.ds(base, b_per_w)], idx_v)

        def remap(k, _):
            v = idx_v[pl.ds(k * 16, 16)]
            j = (
                (v & jnp.int32(~(_BN - 1)))
                | ((v & jnp.int32(_H - 1)) << 1)
                | ((v >> (_H.bit_length() - 1)) & jnp.int32(1))
            )
            idx_v[pl.ds(k * 16, 16)] = j
            return 0

        lax.fori_loop(0, b_per_w // 16, remap, 0)

        bufs = (rows0, rows1)
        gsems = (g0, g1)
        ssems = (s0, s1)

        def gather(ci):
            b = ci % 2
            return pltpu.async_copy(
                table_hbm.at[idx_v.at[pl.ds(ci * C, C)]], bufs[b], gsems[b]
            )

        def scatter(ci):
            b = ci % 2
            return pltpu.async_copy(
                bufs[b], out_hbm.at[pl.ds(base + ci * C, C)], ssems[b]
            )

        gathers = [None] * n_chunks
        scatters = [None] * n_chunks
        gathers[0] = gather(0)
        for ci in range(n_chunks):
            if ci + 1 < n_chunks:
                if ci - 1 >= 0:
                    scatters[ci - 1].wait()
                gathers[ci + 1] = gather(ci + 1)
            gathers[ci].wait()
            scatters[ci] = scatter(ci)
        scatters[n_chunks - 2].wait()
        scatters[n_chunks - 1].wait()

    return emb


def kernel(params, ids):
    V, D = params.shape
    ids_shape = ids.shape
    B = 1
    for s in ids_shape:
        B *= s
    NW = 32
    b_per_w = B // NW
    C = 800
    n_chunks = b_per_w // C

    packed = _transpose_table(params.T, V, D)
    table = packed.reshape(packed.shape[0] * 2, D)
    ids_flat = ids.reshape((B,)).astype(jnp.int32)
    out = _emb_kernel(B, D, b_per_w, C, n_chunks)(table, ids_flat)
    return out.reshape(tuple(ids_shape) + (D,))


# speedup vs baseline: 2.1860x; 1.0778x over previous
import functools

import jax
import jax.numpy as jnp
from jax import lax
from jax.experimental import pallas as pl
from jax.experimental.pallas import tpu as pltpu
from jax.experimental.pallas import tpu_sc as plsc

_BN = 16384
_H = _BN // 2


def _transpose_kernel(x_ref, o_ref):
    x = x_ref[...]
    xx = jnp.concatenate([x[:, :_H], x[:, _H:]], axis=0)
    o_ref[...] = xx.T


def _transpose_table(params_t, V, D):
    grid = pl.cdiv(V, _BN)
    return pl.pallas_call(
        _transpose_kernel,
        grid=(grid,),
        in_specs=[pl.BlockSpec((D, _BN), lambda b: (0, b))],
        out_specs=pl.BlockSpec((_H, 2 * D), lambda b: (b, 0)),
        out_shape=jax.ShapeDtypeStruct((grid * _H, 2 * D), jnp.float32),
        compiler_params=pltpu.CompilerParams(
            vmem_limit_bytes=100 * 1024 * 1024
        ),
    )(params_t)


def _emb_kernel(B, D, b_per_w, C, n_chunks):
    mesh = plsc.VectorSubcoreMesh(core_axis_name="c", subcore_axis_name="s")

    @functools.partial(
        pl.kernel,
        mesh=mesh,
        out_type=jax.ShapeDtypeStruct((B, D), jnp.float32),
        scratch_types=[
            pltpu.VMEM((b_per_w,), jnp.int32),
            pltpu.VMEM((C, D), jnp.float32),
            pltpu.VMEM((C, D), jnp.float32),
            pltpu.SemaphoreType.DMA,
            pltpu.SemaphoreType.DMA,
            pltpu.SemaphoreType.DMA,
            pltpu.SemaphoreType.DMA,
        ],
        compiler_params=pltpu.CompilerParams(use_tc_tiling_on_sc=False),
    )
    def emb(table_hbm, idx_hbm, out_hbm, idx_v, rows0, rows1, g0, g1, s0, s1):
        nc = 2
        wid = lax.axis_index("s") * nc + lax.axis_index("c")
        base = wid * b_per_w
        pltpu.sync_copy(idx_hbm.at[pl.ds(base, b_per_w)], idx_v)

        def remap(k, _):
            v = idx_v[pl.ds(k * 16, 16)]
            j = (
                (v & jnp.int32(~(_BN - 1)))
                | ((v & jnp.int32(_H - 1)) << 1)
                | ((v >> (_H.bit_length() - 1)) & jnp.int32(1))
            )
            idx_v[pl.ds(k * 16, 16)] = j
            return 0

        lax.fori_loop(0, b_per_w // 16, remap, 0)

        bufs = (rows0, rows1)
        gsems = (g0, g1)
        ssems = (s0, s1)

        def gather(ci):
            b = ci % 2
            return pltpu.async_copy(
                table_hbm.at[idx_v.at[pl.ds(ci * C, C)]], bufs[b], gsems[b]
            )

        def scatter(ci):
            b = ci % 2
            return pltpu.async_copy(
                bufs[b], out_hbm.at[pl.ds(base + ci * C, C)], ssems[b]
            )

        gathers = [None] * n_chunks
        scatters = [None] * n_chunks
        gathers[0] = gather(0)
        for ci in range(n_chunks):
            if ci + 1 < n_chunks:
                if ci - 1 >= 0:
                    scatters[ci - 1].wait()
                gathers[ci + 1] = gather(ci + 1)
            gathers[ci].wait()
            scatters[ci] = scatter(ci)
        scatters[n_chunks - 2].wait()
        scatters[n_chunks - 1].wait()

    return emb


def kernel(params, ids):
    V, D = params.shape
    ids_shape = ids.shape
    B = 1
    for s in ids_shape:
        B *= s
    NW = 32
    b_per_w = B // NW
    C = 800
    n_chunks = b_per_w // C

    packed = _transpose_table(params.T, V, D)
    table = packed.reshape(packed.shape[0] * 2, D)
    ids_flat = ids.reshape((B,)).astype(jnp.int32)
    out = _emb_kernel(B, D, b_per_w, C, n_chunks)(table, ids_flat)
    return out.reshape(tuple(ids_shape) + (D,))


# speedup vs baseline: 2.2271x; 1.0188x over previous
import functools

import jax
import jax.numpy as jnp
from jax import lax
from jax.experimental import pallas as pl
from jax.experimental.pallas import tpu as pltpu
from jax.experimental.pallas import tpu_sc as plsc

_BN = 32768
_H = _BN // 2


def _transpose_kernel(x_ref, o_ref):
    x = x_ref[...]
    xx = jnp.concatenate([x[:, :_H], x[:, _H:]], axis=0)
    o_ref[...] = xx.T


def _transpose_table(params_t, V, D):
    grid = pl.cdiv(V, _BN)
    return pl.pallas_call(
        _transpose_kernel,
        grid=(grid,),
        in_specs=[pl.BlockSpec((D, _BN), lambda b: (0, b))],
        out_specs=pl.BlockSpec((_H, 2 * D), lambda b: (b, 0)),
        out_shape=jax.ShapeDtypeStruct((grid * _H, 2 * D), jnp.float32),
        compiler_params=pltpu.CompilerParams(
            vmem_limit_bytes=100 * 1024 * 1024
        ),
    )(params_t)


def _emb_kernel(B, D, b_per_w, C, n_chunks):
    mesh = plsc.VectorSubcoreMesh(core_axis_name="c", subcore_axis_name="s")

    @functools.partial(
        pl.kernel,
        mesh=mesh,
        out_type=jax.ShapeDtypeStruct((B, D), jnp.float32),
        scratch_types=[
            pltpu.VMEM((b_per_w,), jnp.int32),
            pltpu.VMEM((C, D), jnp.float32),
            pltpu.VMEM((C, D), jnp.float32),
            pltpu.SemaphoreType.DMA,
            pltpu.SemaphoreType.DMA,
            pltpu.SemaphoreType.DMA,
            pltpu.SemaphoreType.DMA,
        ],
        compiler_params=pltpu.CompilerParams(use_tc_tiling_on_sc=False),
    )
    def emb(table_hbm, idx_hbm, out_hbm, idx_v, rows0, rows1, g0, g1, s0, s1):
        nc = 2
        wid = lax.axis_index("s") * nc + lax.axis_index("c")
        base = wid * b_per_w
        pltpu.sync_copy(idx_hbm.at[pl.ds(base, b_per_w)], idx_v)

        def remap(k, _):
            v = idx_v[pl.ds(k * 16, 16)]
            j = (
                (v & jnp.int32(~(_BN - 1)))
                | ((v & jnp.int32(_H - 1)) << 1)
                | ((v >> (_H.bit_length() - 1)) & jnp.int32(1))
            )
            idx_v[pl.ds(k * 16, 16)] = j
            return 0

        lax.fori_loop(0, b_per_w // 16, remap, 0)

        bufs = (rows0, rows1)
        gsems = (g0, g1)
        ssems = (s0, s1)

        def gather(ci):
            b = ci % 2
            return pltpu.async_copy(
                table_hbm.at[idx_v.at[pl.ds(ci * C, C)]], bufs[b], gsems[b]
            )

        def scatter(ci):
            b = ci % 2
            return pltpu.async_copy(
                bufs[b], out_hbm.at[pl.ds(base + ci * C, C)], ssems[b]
            )

        gathers = [None] * n_chunks
        scatters = [None] * n_chunks
        gathers[0] = gather(0)
        for ci in range(n_chunks):
            if ci + 1 < n_chunks:
                if ci - 1 >= 0:
                    scatters[ci - 1].wait()
                gathers[ci + 1] = gather(ci + 1)
            gathers[ci].wait()
            scatters[ci] = scatter(ci)
        scatters[n_chunks - 2].wait()
        scatters[n_chunks - 1].wait()

    return emb


def kernel(params, ids):
    V, D = params.shape
    ids_shape = ids.shape
    B = 1
    for s in ids_shape:
        B *= s
    NW = 32
    b_per_w = B // NW
    C = 800
    n_chunks = b_per_w // C

    packed = _transpose_table(params.T, V, D)
    table = packed.reshape(packed.shape[0] * 2, D)
    ids_flat = ids.reshape((B,)).astype(jnp.int32)
    out = _emb_kernel(B, D, b_per_w, C, n_chunks)(table, ids_flat)
    return out.reshape(tuple(ids_shape) + (D,))
